# baseline (speedup 1.0000x reference)
import jax
import jax.numpy as jnp
from jax.experimental import pallas as pl
from jax.experimental.pallas import tpu as pltpu

M = 4096
N = 4096
BI = 256


def _stencil_kernel(thr_ref, top_ref, bot_ref, d_ref, out_ref):
    i = pl.program_id(0)
    t = thr_ref[0]
    c = d_ref[...]
    top = top_ref[0]
    bot = bot_ref[0]

    inf = jnp.float32(jnp.inf)
    infcol = jnp.full((BI + 1, 1), inf, jnp.float32)

    s = jnp.concatenate([top, c, bot], axis=0)
    pn = jnp.concatenate([s[1:, 1:], infcol], axis=1)
    pabs = jnp.abs(pn - s[:-1, :])
    qn = jnp.concatenate([s[:-1, 1:], s[:-1, :1]], axis=1)
    qabs = jnp.abs(qn - s[1:, :])

    cC = pabs[1:, :] <= t
    cD = qabs[:-1, :] <= t
    cA = jnp.concatenate([infcol[1:], pabs[:-1, :-1]], axis=1) <= t
    cB = jnp.concatenate([infcol[1:], qabs[1:, :-1]], axis=1) <= t

    lanes = jax.lax.broadcasted_iota(jnp.int32, (BI, N), 1)
    rows = jax.lax.broadcasted_iota(jnp.int32, (BI, N), 0)
    lmB = lanes <= N - 2
    rm = rows >= 1 - i * BI

    combined = ((cA | (cB & lmB)) & rm) | cC | cD
    out_ref[...] = combined.astype(jnp.int8).T != 0


def kernel(d_noised, threshold):
    nb = M // BI
    starts = jnp.arange(nb) * BI
    inf_row = jnp.full((1, N), jnp.inf, jnp.float32)
    top_rows = jnp.take(d_noised, (starts - 1) % M, axis=0)
    bot_rows = jnp.concatenate(
        [jnp.take(d_noised, starts[:-1] + BI, axis=0), inf_row], axis=0
    )
    thr = jnp.reshape(threshold, (1,))

    out = pl.pallas_call(
        _stencil_kernel,
        grid=(nb,),
        in_specs=[
            pl.BlockSpec(memory_space=pltpu.SMEM),
            pl.BlockSpec((1, 1, N), lambda i: (i, 0, 0)),
            pl.BlockSpec((1, 1, N), lambda i: (i, 0, 0)),
            pl.BlockSpec((BI, N), lambda i: (i, 0)),
        ],
        out_specs=pl.BlockSpec((N, BI), lambda i: (0, i)),
        out_shape=jax.ShapeDtypeStruct((N, M), jnp.bool_),
        compiler_params=pltpu.CompilerParams(
            dimension_semantics=("arbitrary",),
        ),
    )(
        thr,
        top_rows.reshape(nb, 1, N),
        bot_rows.reshape(nb, 1, N),
        d_noised,
    )
    return out


# speedup vs baseline: 1.0079x; 1.0079x over previous
import jax
import jax.numpy as jnp
from jax.experimental import pallas as pl
from jax.experimental.pallas import tpu as pltpu

M = 4096
N = 4096
BI = 256


def _stencil_kernel(thr_ref, top_ref, bot_ref, d_ref, out_ref):
    i = pl.program_id(0)
    t = thr_ref[0]
    c = d_ref[...]
    top = top_ref[0]
    bot = bot_ref[0]

    inf = jnp.float32(jnp.inf)
    infcol = jnp.full((BI, 1), inf, jnp.float32)

    up = jnp.concatenate([top, c[:-1, :]], axis=0)
    down = jnp.concatenate([c[1:, :], bot], axis=0)

    upAL = jnp.concatenate([infcol, up[:, :-1]], axis=1)
    dnL = jnp.concatenate([infcol, down[:, :-1]], axis=1)
    dnRC = jnp.concatenate([down[:, 1:], infcol], axis=1)
    upDR = jnp.concatenate([up[:, 1:], up[:, :1]], axis=1)

    cA = jnp.abs(upAL - c) <= t
    cB = jnp.abs(dnL - c) <= t
    cC = jnp.abs(dnRC - c) <= t
    cD = jnp.abs(upDR - c) <= t

    lanes = jax.lax.broadcasted_iota(jnp.int32, (BI, N), 1)
    rows = jax.lax.broadcasted_iota(jnp.int32, (BI, N), 0)
    lmB = lanes <= N - 2
    rm = rows >= 1 - i * BI

    combined = ((cA | (cB & lmB)) & rm) | cC | cD
    out_ref[...] = combined.astype(jnp.int8).T != 0


def kernel(d_noised, threshold):
    nb = M // BI
    starts = jnp.arange(nb) * BI
    inf_row = jnp.full((1, N), jnp.inf, jnp.float32)
    top_rows = jnp.take(d_noised, (starts - 1) % M, axis=0)
    bot_rows = jnp.concatenate(
        [jnp.take(d_noised, starts[:-1] + BI, axis=0), inf_row], axis=0
    )
    thr = jnp.reshape(threshold, (1,))

    out = pl.pallas_call(
        _stencil_kernel,
        grid=(nb,),
        in_specs=[
            pl.BlockSpec(memory_space=pltpu.SMEM),
            pl.BlockSpec((1, 1, N), lambda i: (i, 0, 0)),
            pl.BlockSpec((1, 1, N), lambda i: (i, 0, 0)),
            pl.BlockSpec((BI, N), lambda i: (i, 0)),
        ],
        out_specs=pl.BlockSpec((N, BI), lambda i: (0, i)),
        out_shape=jax.ShapeDtypeStruct((N, M), jnp.bool_),
        compiler_params=pltpu.CompilerParams(
            dimension_semantics=("arbitrary",),
        ),
    )(
        thr,
        top_rows.reshape(nb, 1, N),
        bot_rows.reshape(nb, 1, N),
        d_noised,
    )
    return out


# speedup vs baseline: 1.0504x; 1.0422x over previous
import jax
import jax.numpy as jnp
from jax.experimental import pallas as pl
from jax.experimental.pallas import tpu as pltpu

M = 4096
N = 4096
BI = 256


def _stencil_kernel(thr_ref, top_ref, bot_ref, d_ref, out_ref):
    i = pl.program_id(0)
    t = thr_ref[0]
    c = d_ref[...]
    top = top_ref[0]
    bot = bot_ref[0]

    inf = jnp.float32(jnp.inf)
    infcol = jnp.full((BI, 1), inf, jnp.float32)

    up = jnp.concatenate([top, c[:-1, :]], axis=0)
    down = jnp.concatenate([c[1:, :], bot], axis=0)

    cR = jnp.concatenate([c[:, 1:], c[:, :1]], axis=1)
    cL = jnp.concatenate([c[:, -1:], c[:, :-1]], axis=1)

    lanes = jax.lax.broadcasted_iota(jnp.int32, (BI, N), 1)
    rows = jax.lax.broadcasted_iota(jnp.int32, (BI, N), 0)

    b_field = jnp.where(lanes == N - 2, inf, jnp.abs(down - cR))
    om1 = jnp.minimum(jnp.abs(up - cR), b_field)
    g1 = jnp.concatenate([infcol, om1[:, :-1]], axis=1)

    c_field = jnp.where(lanes == 0, inf, jnp.abs(down - cL))
    op1 = jnp.minimum(c_field, jnp.abs(up - cL))
    g2 = jnp.concatenate([op1[:, 1:], op1[:, :1]], axis=1)

    rm = rows >= 1 - i * BI
    combined = ((g1 <= t) & rm) | (g2 <= t)
    out_ref[...] = combined.astype(jnp.int8).T != 0


def kernel(d_noised, threshold):
    nb = M // BI
    starts = jnp.arange(nb) * BI
    inf_row = jnp.full((1, N), jnp.inf, jnp.float32)
    top_rows = jnp.take(d_noised, (starts - 1) % M, axis=0)
    bot_rows = jnp.concatenate(
        [jnp.take(d_noised, starts[:-1] + BI, axis=0), inf_row], axis=0
    )
    thr = jnp.reshape(threshold, (1,))

    out = pl.pallas_call(
        _stencil_kernel,
        grid=(nb,),
        in_specs=[
            pl.BlockSpec(memory_space=pltpu.SMEM),
            pl.BlockSpec((1, 1, N), lambda i: (i, 0, 0)),
            pl.BlockSpec((1, 1, N), lambda i: (i, 0, 0)),
            pl.BlockSpec((BI, N), lambda i: (i, 0)),
        ],
        out_specs=pl.BlockSpec((N, BI), lambda i: (0, i)),
        out_shape=jax.ShapeDtypeStruct((N, M), jnp.bool_),
        compiler_params=pltpu.CompilerParams(
            dimension_semantics=("arbitrary",),
        ),
    )(
        thr,
        top_rows.reshape(nb, 1, N),
        bot_rows.reshape(nb, 1, N),
        d_noised,
    )
    return out
